# baseline (speedup 1.0000x reference)
import functools

import jax
import jax.numpy as jnp
from jax import lax
from jax.experimental import pallas as pl
from jax.experimental.pallas import tpu as pltpu
from jax.experimental.pallas import tpu_sc as plsc

NUM_CORES = 2
NUM_SUBCORES = 16
LANES = 16
NW = NUM_CORES * NUM_SUBCORES

BATCH = 16384
EMBED_DIM = 32
B_PER_W = BATCH // NW


def _mf_kernel(user_ids, item_ids, user_table, item_table, user_bias,
               item_bias, global_bias, out_hbm,
               uidx_v, iidx_v, urows_v, irows_v, ub_v, ib_v, gb_v, out_v,
               sem):
    wid = lax.axis_index("s") * NUM_CORES + lax.axis_index("c")
    base = wid * B_PER_W

    pltpu.sync_copy(user_ids.at[pl.ds(base, B_PER_W)], uidx_v)
    pltpu.sync_copy(item_ids.at[pl.ds(base, B_PER_W)], iidx_v)
    pltpu.sync_copy(global_bias, gb_v)

    c1 = pltpu.async_copy(user_table.at[uidx_v], urows_v, sem)
    c2 = pltpu.async_copy(item_table.at[iidx_v], irows_v, sem)
    c3 = pltpu.async_copy(user_bias.at[uidx_v], ub_v, sem)
    c4 = pltpu.async_copy(item_bias.at[iidx_v], ib_v, sem)
    c1.wait()
    c2.wait()
    c3.wait()
    c4.wait()

    gb = gb_v[...]

    def block_body(blk, carry):
        off = blk * LANES
        rows = off + lax.iota(jnp.int32, LANES)
        acc = ub_v[pl.ds(off, LANES)] + ib_v[pl.ds(off, LANES)] + gb
        for d in range(EMBED_DIM):
            cols = jnp.full((LANES,), d, jnp.int32)
            u = plsc.load_gather(urows_v, [rows, cols])
            v = plsc.load_gather(irows_v, [rows, cols])
            acc = acc + u * v
        out_v[pl.ds(off, LANES)] = acc
        return carry

    lax.fori_loop(0, B_PER_W // LANES, block_body, 0, unroll=2)

    pltpu.sync_copy(out_v, out_hbm.at[pl.ds(base, B_PER_W)])


@jax.jit
def kernel(user_ids, item_ids, user_table, item_table, user_bias, item_bias,
           global_bias):
    mesh = plsc.VectorSubcoreMesh(core_axis_name="c", subcore_axis_name="s")
    run = pl.kernel(
        _mf_kernel, mesh=mesh,
        compiler_params=pltpu.CompilerParams(
            needs_layout_passes=False, use_tc_tiling_on_sc=False),
        out_type=jax.ShapeDtypeStruct((BATCH,), jnp.float32),
        scratch_types=[
            pltpu.VMEM((B_PER_W,), jnp.int32),
            pltpu.VMEM((B_PER_W,), jnp.int32),
            pltpu.VMEM((B_PER_W, EMBED_DIM), jnp.float32),
            pltpu.VMEM((B_PER_W, EMBED_DIM), jnp.float32),
            pltpu.VMEM((B_PER_W,), jnp.float32),
            pltpu.VMEM((B_PER_W,), jnp.float32),
            pltpu.VMEM((LANES,), jnp.float32),
            pltpu.VMEM((B_PER_W,), jnp.float32),
            pltpu.SemaphoreType.DMA,
        ],
    )
    gb16 = jnp.broadcast_to(global_bias.astype(jnp.float32), (LANES,))
    return run(user_ids.astype(jnp.int32), item_ids.astype(jnp.int32),
               user_table, item_table,
               user_bias.reshape(-1), item_bias.reshape(-1), gb16)
